# baseline (speedup 1.0000x reference)
import jax
import jax.numpy as jnp
from jax import lax
from jax.experimental import pallas as pl
from jax.experimental.pallas import tpu as pltpu
from jax.experimental.pallas import tpu_sc as plsc

VOCAB = 100000
HIDDEN = 1024
BATCH = 4
SEQ = 2048

TOKENS = BATCH * SEQ
NW = 32
T_PER_W = TOKENS // NW
CH = 32
NCH = T_PER_W // CH
LANES = 16
VECS_PER_CHUNK = CH * HIDDEN // LANES


def _emb_body(x_hbm, pos_hbm, table_hbm, out_hbm,
              idx_v, buf0, buf1, posb, sem0, sem1):
    nc = 2
    wid = lax.axis_index("s") * nc + lax.axis_index("c")
    base = wid * T_PER_W
    s_start = lax.rem(base, SEQ)

    pltpu.sync_copy(x_hbm.at[pl.ds(base, T_PER_W)], idx_v)

    bufs = (buf0, buf1)
    sems = (sem0, sem1)

    pltpu.async_copy(table_hbm.at[idx_v.at[pl.ds(0, CH)]], buf0, sem0)

    for c in range(NCH):
        cur = bufs[c % 2]
        if c + 1 < NCH:
            pltpu.async_copy(
                table_hbm.at[idx_v.at[pl.ds((c + 1) * CH, CH)]],
                bufs[(c + 1) % 2], sems[(c + 1) % 2])

        pltpu.sync_copy(pos_hbm.at[pl.ds(s_start + c * CH, CH)], posb)

        pltpu.make_async_copy(table_hbm.at[idx_v.at[pl.ds(c * CH, CH)]],
                              cur, sems[c % 2]).wait()

        def add_body(i, _):
            r = i // (HIDDEN // LANES)
            col = (i % (HIDDEN // LANES)) * LANES
            cur[r, pl.ds(col, LANES)] = (
                cur[r, pl.ds(col, LANES)] + posb[r, pl.ds(col, LANES)])
            return 0

        lax.fori_loop(0, VECS_PER_CHUNK, add_body, 0)

        pltpu.sync_copy(cur, out_hbm.at[pl.ds(base + c * CH, CH)])


@jax.jit
def kernel(x, token_table, pos_emb):
    xf = x.reshape(TOKENS).astype(jnp.int32)
    pos = pos_emb.reshape(SEQ, HIDDEN)
    mesh = plsc.VectorSubcoreMesh(core_axis_name="c", subcore_axis_name="s")
    out = pl.kernel(
        _emb_body,
        out_type=jax.ShapeDtypeStruct((TOKENS, HIDDEN), jnp.float32),
        mesh=mesh,
        scratch_types=[
            pltpu.VMEM((T_PER_W,), jnp.int32),
            pltpu.VMEM((CH, HIDDEN), jnp.float32),
            pltpu.VMEM((CH, HIDDEN), jnp.float32),
            pltpu.VMEM((CH, HIDDEN), jnp.float32),
            pltpu.SemaphoreType.DMA,
            pltpu.SemaphoreType.DMA,
        ],
    )(xf, pos, token_table)
    return out.reshape(BATCH, SEQ, HIDDEN)


# speedup vs baseline: 2.2924x; 2.2924x over previous
import jax
import jax.numpy as jnp
from jax import lax
from jax.experimental import pallas as pl
from jax.experimental.pallas import tpu as pltpu
from jax.experimental.pallas import tpu_sc as plsc

VOCAB = 100000
HIDDEN = 1024
BATCH = 4
SEQ = 2048

TOKENS = BATCH * SEQ
NW = 32
P_PER_W = SEQ // NW
CH = 32
NPC = P_PER_W // CH
NSTEP = NPC * BATCH
LANES = 16
VPR = HIDDEN // LANES


def _emb_body(x_hbm, pos_hbm, table_hbm, out_hbm,
              idx_v, buf0, buf1, posb, sem0, sem1):
    nc = 2
    wid = lax.axis_index("s") * nc + lax.axis_index("c")
    p0 = wid * P_PER_W

    for b in range(BATCH):
        pltpu.sync_copy(x_hbm.at[pl.ds(b * SEQ + p0, P_PER_W)],
                        idx_v.at[pl.ds(b * P_PER_W, P_PER_W)])

    bufs = (buf0, buf1)
    sems = (sem0, sem1)

    def fire(j):
        pc, b = divmod(j, BATCH)
        pltpu.async_copy(
            table_hbm.at[idx_v.at[pl.ds(b * P_PER_W + pc * CH, CH)]],
            bufs[j % 2], sems[j % 2])

    pltpu.sync_copy(pos_hbm.at[pl.ds(p0, CH)], posb)
    fire(0)

    for j in range(NSTEP):
        pc, b = divmod(j, BATCH)
        cur = bufs[j % 2]
        if j + 1 < NSTEP:
            fire(j + 1)

        pltpu.make_async_copy(
            table_hbm.at[idx_v.at[pl.ds(b * P_PER_W + pc * CH, CH)]],
            cur, sems[j % 2]).wait()

        @plsc.parallel_loop(0, CH * VPR, unroll=8)
        def _(i):
            r = i >> 6
            col = (i & (VPR - 1)) * LANES
            plsc.addupdate(cur.at[r, pl.ds(col, LANES)],
                           posb[r, pl.ds(col, LANES)])

        pltpu.sync_copy(cur, out_hbm.at[pl.ds(b * SEQ + p0 + pc * CH, CH)])

        if b == BATCH - 1 and pc + 1 < NPC:
            pltpu.sync_copy(pos_hbm.at[pl.ds(p0 + (pc + 1) * CH, CH)], posb)


@jax.jit
def kernel(x, token_table, pos_emb):
    xf = x.reshape(TOKENS).astype(jnp.int32)
    pos = pos_emb.reshape(SEQ, HIDDEN)
    mesh = plsc.VectorSubcoreMesh(core_axis_name="c", subcore_axis_name="s",
                                  num_cores=2, num_subcores=16)
    out = pl.kernel(
        _emb_body,
        out_type=jax.ShapeDtypeStruct((TOKENS, HIDDEN), jnp.float32),
        mesh=mesh,
        scratch_types=[
            pltpu.VMEM((BATCH * P_PER_W,), jnp.int32),
            pltpu.VMEM((CH, HIDDEN), jnp.float32),
            pltpu.VMEM((CH, HIDDEN), jnp.float32),
            pltpu.VMEM((CH, HIDDEN), jnp.float32),
            pltpu.SemaphoreType.DMA,
            pltpu.SemaphoreType.DMA,
        ],
    )(xf, pos, token_table)
    return out.reshape(BATCH, SEQ, HIDDEN)


# speedup vs baseline: 2.3649x; 1.0316x over previous
import jax
import jax.numpy as jnp
from jax import lax
from jax.experimental import pallas as pl
from jax.experimental.pallas import tpu as pltpu
from jax.experimental.pallas import tpu_sc as plsc

VOCAB = 100000
HIDDEN = 1024
BATCH = 4
SEQ = 2048

TOKENS = BATCH * SEQ
NW = 32
P_PER_W = SEQ // NW
CH = 32
NPC = P_PER_W // CH
NSTEP = NPC * BATCH
LANES = 16
VPR = HIDDEN // LANES


def _emb_body(x_hbm, pos_hbm, table_hbm, out_hbm,
              idx_v, buf0, buf1, posb, gsem0, gsem1, wsem0, wsem1):
    nc = 2
    wid = lax.axis_index("s") * nc + lax.axis_index("c")
    p0 = wid * P_PER_W

    for b in range(BATCH):
        pltpu.sync_copy(x_hbm.at[pl.ds(b * SEQ + p0, P_PER_W)],
                        idx_v.at[pl.ds(b * P_PER_W, P_PER_W)])

    bufs = (buf0, buf1)
    gsems = (gsem0, gsem1)
    wsems = (wsem0, wsem1)

    def gsrc(j):
        pc, b = divmod(j, BATCH)
        return table_hbm.at[idx_v.at[pl.ds(b * P_PER_W + pc * CH, CH)]]

    def odst(j):
        pc, b = divmod(j, BATCH)
        return out_hbm.at[pl.ds(b * SEQ + p0 + pc * CH, CH)]

    pltpu.sync_copy(pos_hbm.at[pl.ds(p0, CH)], posb)
    pltpu.async_copy(gsrc(0), bufs[0], gsems[0])

    for j in range(NSTEP):
        pc, b = divmod(j, BATCH)
        cur = bufs[j % 2]
        if j + 1 < NSTEP:
            if j >= 1:
                pltpu.make_async_copy(bufs[(j + 1) % 2], odst(j - 1),
                                      wsems[(j + 1) % 2]).wait()
            pltpu.async_copy(gsrc(j + 1), bufs[(j + 1) % 2], gsems[(j + 1) % 2])

        pltpu.make_async_copy(gsrc(j), cur, gsems[j % 2]).wait()

        @plsc.parallel_loop(0, CH * VPR, unroll=8)
        def _(i):
            r = i >> 6
            col = (i & (VPR - 1)) * LANES
            plsc.addupdate(cur.at[r, pl.ds(col, LANES)],
                           posb[r, pl.ds(col, LANES)])

        pltpu.async_copy(cur, odst(j), wsems[j % 2])

        if b == BATCH - 1 and pc + 1 < NPC:
            pltpu.sync_copy(pos_hbm.at[pl.ds(p0 + (pc + 1) * CH, CH)], posb)

    pltpu.make_async_copy(bufs[(NSTEP - 1) % 2], odst(NSTEP - 1),
                          wsems[(NSTEP - 1) % 2]).wait()
    pltpu.make_async_copy(bufs[(NSTEP - 2) % 2], odst(NSTEP - 2),
                          wsems[(NSTEP - 2) % 2]).wait()


@jax.jit
def kernel(x, token_table, pos_emb):
    pos = pos_emb.reshape(SEQ, HIDDEN)
    mesh = plsc.VectorSubcoreMesh(core_axis_name="c", subcore_axis_name="s",
                                  num_cores=2, num_subcores=16)
    out = pl.kernel(
        _emb_body,
        out_type=jax.ShapeDtypeStruct((TOKENS, HIDDEN), jnp.float32),
        mesh=mesh,
        scratch_types=[
            pltpu.VMEM((BATCH * P_PER_W,), jnp.int32),
            pltpu.VMEM((CH, HIDDEN), jnp.float32),
            pltpu.VMEM((CH, HIDDEN), jnp.float32),
            pltpu.VMEM((CH, HIDDEN), jnp.float32),
            pltpu.SemaphoreType.DMA,
            pltpu.SemaphoreType.DMA,
            pltpu.SemaphoreType.DMA,
            pltpu.SemaphoreType.DMA,
        ],
    )(x.reshape(TOKENS).astype(jnp.int32), pos, token_table)
    return out.reshape(BATCH, SEQ, HIDDEN)


# speedup vs baseline: 2.4394x; 1.0315x over previous
import jax
import jax.numpy as jnp
from jax import lax
from jax.experimental import pallas as pl
from jax.experimental.pallas import tpu as pltpu
from jax.experimental.pallas import tpu_sc as plsc

VOCAB = 100000
HIDDEN = 1024
BATCH = 4
SEQ = 2048

TOKENS = BATCH * SEQ
NW = 32
P_PER_W = SEQ // NW
CH = 16
NPC = P_PER_W // CH
NSTEP = NPC * BATCH
NBUF = 4
AHEAD = 2
LANES = 16
VPR = HIDDEN // LANES


def _emb_body(x_hbm, pos_hbm, table_hbm, out_hbm,
              idx_v, buf0, buf1, buf2, buf3, posb0, posb1,
              g0, g1, g2, g3, w0, w1, w2, w3, psem):
    nc = 2
    wid = lax.axis_index("s") * nc + lax.axis_index("c")
    p0 = wid * P_PER_W

    bufs = (buf0, buf1, buf2, buf3)
    gsems = (g0, g1, g2, g3)
    wsems = (w0, w1, w2, w3)
    posbs = (posb0, posb1)

    for b in range(BATCH):
        pltpu.sync_copy(x_hbm.at[pl.ds(b * SEQ + p0, P_PER_W)],
                        idx_v.at[pl.ds(b * P_PER_W, P_PER_W)])

    def gsrc(j):
        pc, b = divmod(j, BATCH)
        return table_hbm.at[idx_v.at[pl.ds(b * P_PER_W + pc * CH, CH)]]

    def odst(j):
        pc, b = divmod(j, BATCH)
        return out_hbm.at[pl.ds(b * SEQ + p0 + pc * CH, CH)]

    def fire_pos(pc):
        pltpu.async_copy(pos_hbm.at[pl.ds(p0 + pc * CH, CH)],
                         posbs[pc % 2], psem)

    fire_pos(0)
    for j in range(AHEAD):
        pltpu.async_copy(gsrc(j), bufs[j % NBUF], gsems[j % NBUF])

    for j in range(NSTEP):
        pc, b = divmod(j, BATCH)
        cur = bufs[j % NBUF]

        if j + AHEAD < NSTEP:
            k = j + AHEAD
            if k >= NBUF:
                pltpu.make_async_copy(bufs[k % NBUF], odst(k - NBUF),
                                      wsems[k % NBUF]).wait()
            pltpu.async_copy(gsrc(k), bufs[k % NBUF], gsems[k % NBUF])

        if b == 0:
            pltpu.make_async_copy(pos_hbm.at[pl.ds(p0 + pc * CH, CH)],
                                  posbs[pc % 2], psem).wait()

        pltpu.make_async_copy(gsrc(j), cur, gsems[j % NBUF]).wait()

        posb = posbs[pc % 2]

        @plsc.parallel_loop(0, CH * VPR, unroll=8)
        def _(i):
            r = i >> 6
            col = (i & (VPR - 1)) * LANES
            plsc.addupdate(cur.at[r, pl.ds(col, LANES)],
                           posb[r, pl.ds(col, LANES)])

        pltpu.async_copy(cur, odst(j), wsems[j % NBUF])

        if b == BATCH - 1 and pc + 1 < NPC:
            fire_pos(pc + 1)

    for j in range(NSTEP - NBUF, NSTEP):
        pltpu.make_async_copy(bufs[j % NBUF], odst(j),
                              wsems[j % NBUF]).wait()


@jax.jit
def kernel(x, token_table, pos_emb):
    pos = pos_emb.reshape(SEQ, HIDDEN)
    mesh = plsc.VectorSubcoreMesh(core_axis_name="c", subcore_axis_name="s",
                                  num_cores=2, num_subcores=16)
    out = pl.kernel(
        _emb_body,
        out_type=jax.ShapeDtypeStruct((TOKENS, HIDDEN), jnp.float32),
        mesh=mesh,
        scratch_types=[
            pltpu.VMEM((BATCH * P_PER_W,), jnp.int32),
            pltpu.VMEM((CH, HIDDEN), jnp.float32),
            pltpu.VMEM((CH, HIDDEN), jnp.float32),
            pltpu.VMEM((CH, HIDDEN), jnp.float32),
            pltpu.VMEM((CH, HIDDEN), jnp.float32),
            pltpu.VMEM((CH, HIDDEN), jnp.float32),
            pltpu.VMEM((CH, HIDDEN), jnp.float32),
            pltpu.SemaphoreType.DMA,
            pltpu.SemaphoreType.DMA,
            pltpu.SemaphoreType.DMA,
            pltpu.SemaphoreType.DMA,
            pltpu.SemaphoreType.DMA,
            pltpu.SemaphoreType.DMA,
            pltpu.SemaphoreType.DMA,
            pltpu.SemaphoreType.DMA,
            pltpu.SemaphoreType.DMA,
        ],
    )(x.reshape(TOKENS).astype(jnp.int32), pos, token_table)
    return out.reshape(BATCH, SEQ, HIDDEN)


# speedup vs baseline: 2.4870x; 1.0195x over previous
import jax
import jax.numpy as jnp
from jax import lax
from jax.experimental import pallas as pl
from jax.experimental.pallas import tpu as pltpu
from jax.experimental.pallas import tpu_sc as plsc

VOCAB = 100000
HIDDEN = 1024
BATCH = 4
SEQ = 2048

TOKENS = BATCH * SEQ
NW = 32
P_PER_W = SEQ // NW
CH = 16
NPC = P_PER_W // CH
NSTEP = NPC * BATCH
NBUF = 5
AHEAD = 3
LANES = 16
VPR = HIDDEN // LANES


def _emb_body(x_hbm, pos_hbm, table_hbm, out_hbm,
              idx_v, buf0, buf1, buf2, buf3, buf4, posb0, posb1,
              g0, g1, g2, g3, g4, w0, w1, w2, w3, w4, psem, isem):
    nc = 2
    wid = lax.axis_index("s") * nc + lax.axis_index("c")
    p0 = wid * P_PER_W

    bufs = (buf0, buf1, buf2, buf3, buf4)
    gsems = (g0, g1, g2, g3, g4)
    wsems = (w0, w1, w2, w3, w4)
    posbs = (posb0, posb1)

    idx_copies = []
    for b in range(BATCH):
        idx_copies.append(pltpu.make_async_copy(
            x_hbm.at[pl.ds(b * SEQ + p0, P_PER_W)],
            idx_v.at[pl.ds(b * P_PER_W, P_PER_W)], isem))
        idx_copies[-1].start()
    for cp in idx_copies:
        cp.wait()

    def gsrc(j):
        pc, b = divmod(j, BATCH)
        return table_hbm.at[idx_v.at[pl.ds(b * P_PER_W + pc * CH, CH)]]

    def odst(j):
        pc, b = divmod(j, BATCH)
        return out_hbm.at[pl.ds(b * SEQ + p0 + pc * CH, CH)]

    def fire_pos(pc):
        pltpu.async_copy(pos_hbm.at[pl.ds(p0 + pc * CH, CH)],
                         posbs[pc % 2], psem)

    fire_pos(0)
    for j in range(AHEAD):
        pltpu.async_copy(gsrc(j), bufs[j % NBUF], gsems[j % NBUF])

    for j in range(NSTEP):
        pc, b = divmod(j, BATCH)
        cur = bufs[j % NBUF]

        if j + AHEAD < NSTEP:
            k = j + AHEAD
            if k >= NBUF:
                pltpu.make_async_copy(bufs[k % NBUF], odst(k - NBUF),
                                      wsems[k % NBUF]).wait()
            pltpu.async_copy(gsrc(k), bufs[k % NBUF], gsems[k % NBUF])

        if b == 0:
            pltpu.make_async_copy(pos_hbm.at[pl.ds(p0 + pc * CH, CH)],
                                  posbs[pc % 2], psem).wait()

        pltpu.make_async_copy(gsrc(j), cur, gsems[j % NBUF]).wait()

        posb = posbs[pc % 2]

        @plsc.parallel_loop(0, CH * VPR, unroll=8)
        def _(i):
            r = i >> 6
            col = (i & (VPR - 1)) * LANES
            plsc.addupdate(cur.at[r, pl.ds(col, LANES)],
                           posb[r, pl.ds(col, LANES)])

        pltpu.async_copy(cur, odst(j), wsems[j % NBUF])

        if b == BATCH - 1 and pc + 1 < NPC:
            fire_pos(pc + 1)

    for j in range(NSTEP - NBUF, NSTEP):
        pltpu.make_async_copy(bufs[j % NBUF], odst(j),
                              wsems[j % NBUF]).wait()


@jax.jit
def kernel(x, token_table, pos_emb):
    pos = pos_emb.reshape(SEQ, HIDDEN)
    mesh = plsc.VectorSubcoreMesh(core_axis_name="c", subcore_axis_name="s",
                                  num_cores=2, num_subcores=16)
    out = pl.kernel(
        _emb_body,
        out_type=jax.ShapeDtypeStruct((TOKENS, HIDDEN), jnp.float32),
        mesh=mesh,
        scratch_types=[
            pltpu.VMEM((BATCH * P_PER_W,), jnp.int32),
            pltpu.VMEM((CH, HIDDEN), jnp.float32),
            pltpu.VMEM((CH, HIDDEN), jnp.float32),
            pltpu.VMEM((CH, HIDDEN), jnp.float32),
            pltpu.VMEM((CH, HIDDEN), jnp.float32),
            pltpu.VMEM((CH, HIDDEN), jnp.float32),
            pltpu.VMEM((CH, HIDDEN), jnp.float32),
            pltpu.VMEM((CH, HIDDEN), jnp.float32),
            pltpu.SemaphoreType.DMA,
            pltpu.SemaphoreType.DMA,
            pltpu.SemaphoreType.DMA,
            pltpu.SemaphoreType.DMA,
            pltpu.SemaphoreType.DMA,
            pltpu.SemaphoreType.DMA,
            pltpu.SemaphoreType.DMA,
            pltpu.SemaphoreType.DMA,
            pltpu.SemaphoreType.DMA,
            pltpu.SemaphoreType.DMA,
            pltpu.SemaphoreType.DMA,
            pltpu.SemaphoreType.DMA,
        ],
    )(x.reshape(TOKENS).astype(jnp.int32), pos, token_table)
    return out.reshape(BATCH, SEQ, HIDDEN)
